# baseline (speedup 1.0000x reference)
import functools

import jax
import jax.numpy as jnp
from jax import lax
from jax.experimental import pallas as pl
from jax.experimental.pallas import tpu as pltpu
from jax.experimental.pallas import tpu_sc as plsc

N_NODES = 10000
N_EDGES = 160000
IN_FEATS = 256
H_FEATS = 1024
NUM_CLASSES = 128

NPAD = 10240
NTILE = NPAD // 16
HF = IN_FEATS // 2
EPT = N_EDGES // 16
CH = 96
NCH = 105
EPP = NCH * CH

_f32 = jnp.float32
_i32 = jnp.int32


def _rsqrt16(d):
    i = plsc.bitcast(d, _i32)
    i = 0x5F3759DF - lax.shift_right_logical(i, 1)
    y = plsc.bitcast(i, _f32)
    for _ in range(3):
        y = y * (1.5 - 0.5 * d * y * y)
    return jnp.where(d > 0.0, y, 0.0)


def _deg_norm_call(src, dst):
    mesh = plsc.VectorSubcoreMesh(core_axis_name="c", subcore_axis_name="s")

    @functools.partial(
        pl.kernel,
        mesh=mesh,
        out_type=[
            jax.ShapeDtypeStruct((NPAD,), _f32),
            jax.ShapeDtypeStruct((NPAD,), _f32),
        ],
        compiler_params=pltpu.CompilerParams(needs_layout_passes=False),
        scratch_types=[
            pltpu.VMEM((EPT,), _i32),
            pltpu.VMEM((NPAD,), _f32),
            pltpu.VMEM((NTILE,), _f32),
            pltpu.VMEM((NTILE,), _f32),
            pltpu.VMEM_SHARED((16, NPAD), _f32),
        ],
    )
    def deg_kernel(src_h, dst_h, nout_h, nin_h, idx_v, hist_v, tmp_v, acc_v,
                   stage):
        cid = lax.axis_index("c")
        sid = lax.axis_index("s")
        base = sid * EPT
        nb = sid * NTILE

        @pl.when(cid == 0)
        def _():
            pltpu.sync_copy(src_h.at[pl.ds(base, EPT)], idx_v)

        @pl.when(cid == 1)
        def _():
            pltpu.sync_copy(dst_h.at[pl.ds(base, EPT)], idx_v)

        z16 = jnp.zeros((16,), _f32)
        ones16 = jnp.ones((16,), _f32)

        def zbody(i, c):
            hist_v[pl.ds(i * 16, 16)] = z16
            return c

        def ebody(i, c):
            idx = idx_v[pl.ds(i * 16, 16)]
            plsc.addupdate_scatter(hist_v, [idx], ones16)
            return c

        def abody(t, c):
            sl = pl.ds(t * 16, 16)
            acc_v[sl] = acc_v[sl] + tmp_v[sl]
            return c

        lax.fori_loop(0, NPAD // 16, zbody, 0)
        lax.fori_loop(0, EPT // 16, ebody, 0)

        pltpu.sync_copy(hist_v, stage.at[sid])
        plsc.subcore_barrier()

        pltpu.sync_copy(stage.at[0, pl.ds(nb, NTILE)], acc_v)
        for k in range(1, 16):
            pltpu.sync_copy(stage.at[k, pl.ds(nb, NTILE)], tmp_v)
            lax.fori_loop(0, NTILE // 16, abody, 0)

        def rbody(t, c):
            sl = pl.ds(t * 16, 16)
            acc_v[sl] = _rsqrt16(acc_v[sl])
            return c

        lax.fori_loop(0, NTILE // 16, rbody, 0)

        @pl.when(cid == 0)
        def _():
            pltpu.sync_copy(acc_v, nout_h.at[pl.ds(nb, NTILE)])

        @pl.when(cid == 1)
        def _():
            pltpu.sync_copy(acc_v, nin_h.at[pl.ds(nb, NTILE)])

    return deg_kernel(src, dst)


def _u_call(src, dst, norm_in):
    EPT2 = N_EDGES // 32
    mesh = plsc.VectorSubcoreMesh(core_axis_name="c", subcore_axis_name="s")

    @functools.partial(
        pl.kernel,
        mesh=mesh,
        out_type=jax.ShapeDtypeStruct((2, NPAD), _f32),
        compiler_params=pltpu.CompilerParams(needs_layout_passes=False),
        scratch_types=[
            pltpu.VMEM((EPT2,), _i32),
            pltpu.VMEM((EPT2,), _i32),
            pltpu.VMEM((NPAD,), _f32),
            pltpu.VMEM((NPAD,), _f32),
            pltpu.VMEM((NTILE,), _f32),
            pltpu.VMEM((NTILE,), _f32),
            pltpu.VMEM_SHARED((16, NPAD), _f32),
        ],
    )
    def u_kernel(src_h, dst_h, nin_h, u_h, sv_v, dv_v, hist_v, normin_v,
                 tmp_v, acc_v, stage):
        cid = lax.axis_index("c")
        sid = lax.axis_index("s")
        base = (cid * 16 + sid) * EPT2
        nb = sid * NTILE

        pltpu.sync_copy(src_h.at[pl.ds(base, EPT2)], sv_v)
        pltpu.sync_copy(dst_h.at[pl.ds(base, EPT2)], dv_v)
        pltpu.sync_copy(nin_h, normin_v)

        z16 = jnp.zeros((16,), _f32)

        def zbody(i, c):
            hist_v[pl.ds(i * 16, 16)] = z16
            return c

        lax.fori_loop(0, NPAD // 16, zbody, 0)

        def ubody(i, c):
            sl = pl.ds(i * 16, 16)
            vals = plsc.load_gather(normin_v, [dv_v[sl]])
            plsc.addupdate_scatter(hist_v, [sv_v[sl]], vals)
            return c

        lax.fori_loop(0, EPT2 // 16, ubody, 0)

        def abody(t, c):
            sl = pl.ds(t * 16, 16)
            acc_v[sl] = acc_v[sl] + tmp_v[sl]
            return c

        pltpu.sync_copy(hist_v, stage.at[sid])
        plsc.subcore_barrier()
        pltpu.sync_copy(stage.at[0, pl.ds(nb, NTILE)], acc_v)
        for k in range(1, 16):
            pltpu.sync_copy(stage.at[k, pl.ds(nb, NTILE)], tmp_v)
            lax.fori_loop(0, NTILE // 16, abody, 0)

        @pl.when(cid == 0)
        def _():
            pltpu.sync_copy(acc_v, u_h.at[0, pl.ds(nb, NTILE)])

        @pl.when(cid == 1)
        def _():
            pltpu.sync_copy(acc_v, u_h.at[1, pl.ds(nb, NTILE)])

    return u_kernel(src, dst, norm_in)


def _scale_call(x, norm_out):
    mt = 1000

    def body(x_ref, n_ref, o_ref):
        sc = x_ref[...] * n_ref[...]
        o_ref[0] = sc[:, :HF]
        o_ref[1] = sc[:, HF:]

    return pl.pallas_call(
        body,
        grid=(N_NODES // mt,),
        in_specs=[
            pl.BlockSpec((mt, IN_FEATS), lambda m: (m, 0)),
            pl.BlockSpec((mt, 1), lambda m: (m, 0)),
        ],
        out_specs=pl.BlockSpec((2, mt, HF), lambda m: (0, m, 0)),
        out_shape=jax.ShapeDtypeStruct((2, N_NODES, HF), _f32),
    )(x, norm_out)


def _edge_agg_call(srcf, dst, h0v):
    mesh = plsc.VectorSubcoreMesh(core_axis_name="c", subcore_axis_name="s")

    @functools.partial(
        pl.kernel,
        mesh=mesh,
        out_type=jax.ShapeDtypeStruct((2, NPAD, HF), _f32),
        compiler_params=pltpu.CompilerParams(needs_layout_passes=False),
        scratch_types=[
            pltpu.VMEM((EPP,), _i32),
            pltpu.VMEM((EPP,), _i32),
            pltpu.VMEM((CH,), _i32),
            pltpu.VMEM((CH,), _i32),
            pltpu.VMEM((CH,), _i32),
            pltpu.VMEM((CH, HF), _f32),
            pltpu.VMEM((CH, HF), _f32),
            pltpu.VMEM_SHARED((NPAD, HF), _f32),
            pltpu.SemaphoreType.DMA,
            pltpu.SemaphoreType.DMA,
        ],
    )
    def agg_kernel(srcf_h, dst_h, h0v_h, agg_h,
                   g_v, dst_v, idx_a, idx_b, dsts_v, rows_a, rows_b,
                   acc_s, sem_a, sem_b):
        cid = lax.axis_index("c")
        sid = lax.axis_index("s")
        base = sid * EPP
        nb = sid * NTILE
        coff = cid * N_NODES

        pltpu.sync_copy(srcf_h.at[pl.ds(base, EPP)], g_v)
        pltpu.sync_copy(dst_h.at[pl.ds(base, EPP)], dst_v)

        z16 = jnp.zeros((16,), _f32)

        def zrow(i, c):
            for t in range(HF // 16):
                rows_a[i, pl.ds(t * 16, 16)] = z16
            return c

        lax.fori_loop(0, CH, zrow, 0)
        for q in range(NTILE // CH):
            pltpu.sync_copy(rows_a, acc_s.at[pl.ds(nb + q * CH, CH)])
        rem = NTILE - (NTILE // CH) * CH
        if rem:
            pltpu.sync_copy(rows_a.at[pl.ds(0, rem)],
                            acc_s.at[pl.ds(nb + (NTILE // CH) * CH, rem)])

        plsc.subcore_barrier()

        def build(flat, j, dref, off):
            for t in range(CH // 16):
                sl = pl.ds(t * 16, 16)
                dref[sl] = flat[pl.ds(j * CH + t * 16, 16)] + off

        def gather(j, idx, rows, sem):
            build(g_v, j, idx, coff)
            pltpu.async_copy(h0v_h.at[idx], rows, sem)

        def wait(rows, sem):
            pltpu.make_async_copy(h0v_h.at[idx_a], rows, sem).wait()

        def scatter(j, rows):
            build(dst_v, j, dsts_v, 0)
            pltpu.sync_copy(rows, acc_s.at[dsts_v], add=True)

        gather(0, idx_a, rows_a, sem_a)
        gather(1, idx_b, rows_b, sem_b)

        def pair_body(j2, c):
            ja = 2 * j2
            jb = ja + 1
            wait(rows_a, sem_a)
            scatter(ja, rows_a)
            gather(ja + 2, idx_a, rows_a, sem_a)
            wait(rows_b, sem_b)
            scatter(jb, rows_b)

            @pl.when(j2 < (NCH - 3) // 2)
            def _():
                gather(jb + 2, idx_b, rows_b, sem_b)

            return c

        lax.fori_loop(0, (NCH - 1) // 2, pair_body, 0)
        wait(rows_a, sem_a)
        scatter(NCH - 1, rows_a)

        plsc.subcore_barrier()

        @pl.when(cid == 0)
        def _():
            pltpu.sync_copy(acc_s.at[pl.ds(nb, NTILE)],
                            agg_h.at[0, pl.ds(nb, NTILE)])

        @pl.when(cid == 1)
        def _():
            pltpu.sync_copy(acc_s.at[pl.ds(nb, NTILE)],
                            agg_h.at[1, pl.ds(nb, NTILE)])

    return agg_kernel(srcf, dst, h0v)


def _dense_call(agg, w1r, b1, norm_in, u, norm_out, W2, b2):
    mt = 1024
    grid = NPAD // mt
    dn = (((1,), (0,)), ((), ()))
    dn_pool = (((0,), (0,)), ((), ()))
    prec = jax.lax.Precision.DEFAULT

    def body(agg_ref, w1_ref, b1_ref, nin_ref, u_ref, nout_ref, w2_ref,
             b2_ref, out_ref, pooled_ref):
        m = pl.program_id(0)

        @pl.when(m == 0)
        def _():
            pooled_ref[...] = jnp.zeros_like(pooled_ref)

        hid = lax.dot_general(agg_ref[0], w1_ref[0], dn, precision=prec,
                              preferred_element_type=_f32)
        hid = hid + lax.dot_general(agg_ref[1], w1_ref[1], dn, precision=prec,
                                    preferred_element_type=_f32)
        hid = hid * nin_ref[...] + b1_ref[...]
        h1 = jnp.maximum(hid, 0.0)
        w = (u_ref[0] + u_ref[1]) * nout_ref[...]
        pooled_ref[...] += lax.dot_general(w, h1, dn_pool, precision=prec,
                                           preferred_element_type=_f32)

        @pl.when(m == grid - 1)
        def _():
            out_ref[...] = lax.dot_general(
                pooled_ref[...] * (1.0 / N_NODES), w2_ref[...], dn,
                precision=prec, preferred_element_type=_f32) + b2_ref[...]

    return pl.pallas_call(
        body,
        grid=(grid,),
        in_specs=[
            pl.BlockSpec((2, mt, HF), lambda m: (0, m, 0)),
            pl.BlockSpec((2, HF, H_FEATS), lambda m: (0, 0, 0)),
            pl.BlockSpec((1, H_FEATS), lambda m: (0, 0)),
            pl.BlockSpec((mt, 1), lambda m: (m, 0)),
            pl.BlockSpec((2, mt, 1), lambda m: (0, m, 0)),
            pl.BlockSpec((mt, 1), lambda m: (m, 0)),
            pl.BlockSpec((H_FEATS, NUM_CLASSES), lambda m: (0, 0)),
            pl.BlockSpec((1, NUM_CLASSES), lambda m: (0, 0)),
        ],
        out_specs=pl.BlockSpec((1, NUM_CLASSES), lambda m: (0, 0)),
        out_shape=jax.ShapeDtypeStruct((1, NUM_CLASSES), _f32),
        scratch_shapes=[pltpu.VMEM((1, H_FEATS), _f32)],
    )(agg, w1r, b1, norm_in, u, norm_out, W2, b2)


def kernel(x, edge_index, W1, b1, W2, b2):
    src = edge_index[0]
    dst = edge_index[1]

    norm_out_p, norm_in_p = _deg_norm_call(src, dst)
    u2 = _u_call(src, dst, norm_in_p)

    h0 = _scale_call(x, norm_out_p[:N_NODES, None])
    h0v = h0.reshape(2 * N_NODES, HF)

    pad = EPP - EPT
    spad = (jnp.arange(pad, dtype=_i32) * 125) % N_NODES
    dpad = N_NODES + (jnp.arange(pad, dtype=_i32) % (NPAD - N_NODES))
    srcp = jnp.concatenate(
        [src.reshape(16, EPT), jnp.broadcast_to(spad, (16, pad))],
        axis=1).reshape(-1)
    dstp = jnp.concatenate(
        [dst.reshape(16, EPT), jnp.broadcast_to(dpad, (16, pad))],
        axis=1).reshape(-1)
    agg = _edge_agg_call(srcp, dstp, h0v)

    w1r = W1.reshape(2, HF, H_FEATS)
    out = _dense_call(agg, w1r, b1.reshape(1, H_FEATS),
                      norm_in_p[:, None], u2[:, :, None], norm_out_p[:, None],
                      W2, b2.reshape(1, NUM_CLASSES))
    return out.reshape(NUM_CLASSES)


# speedup vs baseline: 1.0058x; 1.0058x over previous
import functools

import jax
import jax.numpy as jnp
from jax import lax
from jax.experimental import pallas as pl
from jax.experimental.pallas import tpu as pltpu
from jax.experimental.pallas import tpu_sc as plsc

N_NODES = 10000
N_EDGES = 160000
IN_FEATS = 256
H_FEATS = 1024
NUM_CLASSES = 128

NPAD = 10240
NTILE = NPAD // 16
HF = IN_FEATS // 2
EPT = N_EDGES // 16
CH = 96
NCH = 105
EPP = NCH * CH

_f32 = jnp.float32
_i32 = jnp.int32


def _rsqrt16(d):
    i = plsc.bitcast(d, _i32)
    i = 0x5F3759DF - lax.shift_right_logical(i, 1)
    y = plsc.bitcast(i, _f32)
    for _ in range(3):
        y = y * (1.5 - 0.5 * d * y * y)
    return jnp.where(d > 0.0, y, 0.0)


def _deg_norm_call(src, dst):
    mesh = plsc.VectorSubcoreMesh(core_axis_name="c", subcore_axis_name="s")

    @functools.partial(
        pl.kernel,
        mesh=mesh,
        out_type=[
            jax.ShapeDtypeStruct((NPAD,), _f32),
            jax.ShapeDtypeStruct((NPAD,), _f32),
        ],
        compiler_params=pltpu.CompilerParams(needs_layout_passes=False),
        scratch_types=[
            pltpu.VMEM((EPT,), _i32),
            pltpu.VMEM((NPAD,), _f32),
            pltpu.VMEM((NTILE,), _f32),
            pltpu.VMEM((NTILE,), _f32),
            pltpu.VMEM_SHARED((16, NPAD), _f32),
        ],
    )
    def deg_kernel(src_h, dst_h, nout_h, nin_h, idx_v, hist_v, tmp_v, acc_v,
                   stage):
        cid = lax.axis_index("c")
        sid = lax.axis_index("s")
        base = sid * EPT
        nb = sid * NTILE

        @pl.when(cid == 0)
        def _():
            pltpu.sync_copy(src_h.at[pl.ds(base, EPT)], idx_v)

        @pl.when(cid == 1)
        def _():
            pltpu.sync_copy(dst_h.at[pl.ds(base, EPT)], idx_v)

        z16 = jnp.zeros((16,), _f32)
        ones16 = jnp.ones((16,), _f32)

        def zbody(i, c):
            hist_v[pl.ds(i * 16, 16)] = z16
            return c

        def ebody(i, c):
            idx = idx_v[pl.ds(i * 16, 16)]
            plsc.addupdate_scatter(hist_v, [idx], ones16)
            return c

        def abody(t, c):
            sl = pl.ds(t * 16, 16)
            acc_v[sl] = acc_v[sl] + tmp_v[sl]
            return c

        lax.fori_loop(0, NPAD // 16, zbody, 0)
        lax.fori_loop(0, EPT // 16, ebody, 0)

        pltpu.sync_copy(hist_v, stage.at[sid])
        plsc.subcore_barrier()

        pltpu.sync_copy(stage.at[0, pl.ds(nb, NTILE)], acc_v)
        for k in range(1, 16):
            pltpu.sync_copy(stage.at[k, pl.ds(nb, NTILE)], tmp_v)
            lax.fori_loop(0, NTILE // 16, abody, 0)

        def rbody(t, c):
            sl = pl.ds(t * 16, 16)
            acc_v[sl] = _rsqrt16(acc_v[sl])
            return c

        lax.fori_loop(0, NTILE // 16, rbody, 0)

        @pl.when(cid == 0)
        def _():
            pltpu.sync_copy(acc_v, nout_h.at[pl.ds(nb, NTILE)])

        @pl.when(cid == 1)
        def _():
            pltpu.sync_copy(acc_v, nin_h.at[pl.ds(nb, NTILE)])

    return deg_kernel(src, dst)


def _u_call(src, dst, norm_in):
    EPT2 = N_EDGES // 32
    mesh = plsc.VectorSubcoreMesh(core_axis_name="c", subcore_axis_name="s")

    @functools.partial(
        pl.kernel,
        mesh=mesh,
        out_type=jax.ShapeDtypeStruct((2, NPAD), _f32),
        compiler_params=pltpu.CompilerParams(needs_layout_passes=False),
        scratch_types=[
            pltpu.VMEM((EPT2,), _i32),
            pltpu.VMEM((EPT2,), _i32),
            pltpu.VMEM((NPAD,), _f32),
            pltpu.VMEM((NPAD,), _f32),
            pltpu.VMEM((NTILE,), _f32),
            pltpu.VMEM((NTILE,), _f32),
            pltpu.VMEM_SHARED((16, NPAD), _f32),
        ],
    )
    def u_kernel(src_h, dst_h, nin_h, u_h, sv_v, dv_v, hist_v, normin_v,
                 tmp_v, acc_v, stage):
        cid = lax.axis_index("c")
        sid = lax.axis_index("s")
        base = (cid * 16 + sid) * EPT2
        nb = sid * NTILE

        pltpu.sync_copy(src_h.at[pl.ds(base, EPT2)], sv_v)
        pltpu.sync_copy(dst_h.at[pl.ds(base, EPT2)], dv_v)
        pltpu.sync_copy(nin_h, normin_v)

        z16 = jnp.zeros((16,), _f32)

        def zbody(i, c):
            hist_v[pl.ds(i * 16, 16)] = z16
            return c

        lax.fori_loop(0, NPAD // 16, zbody, 0)

        def ubody(i, c):
            sl = pl.ds(i * 16, 16)
            vals = plsc.load_gather(normin_v, [dv_v[sl]])
            plsc.addupdate_scatter(hist_v, [sv_v[sl]], vals)
            return c

        lax.fori_loop(0, EPT2 // 16, ubody, 0)

        def abody(t, c):
            sl = pl.ds(t * 16, 16)
            acc_v[sl] = acc_v[sl] + tmp_v[sl]
            return c

        pltpu.sync_copy(hist_v, stage.at[sid])
        plsc.subcore_barrier()
        pltpu.sync_copy(stage.at[0, pl.ds(nb, NTILE)], acc_v)
        for k in range(1, 16):
            pltpu.sync_copy(stage.at[k, pl.ds(nb, NTILE)], tmp_v)
            lax.fori_loop(0, NTILE // 16, abody, 0)

        @pl.when(cid == 0)
        def _():
            pltpu.sync_copy(acc_v, u_h.at[0, pl.ds(nb, NTILE)])

        @pl.when(cid == 1)
        def _():
            pltpu.sync_copy(acc_v, u_h.at[1, pl.ds(nb, NTILE)])

    return u_kernel(src, dst, norm_in)


def _scale_call(x, norm_out):
    mt = 1000

    def body(x_ref, n_ref, o_ref):
        sc = x_ref[...] * n_ref[...]
        o_ref[0] = sc[:, :HF]
        o_ref[1] = sc[:, HF:]

    return pl.pallas_call(
        body,
        grid=(N_NODES // mt,),
        in_specs=[
            pl.BlockSpec((mt, IN_FEATS), lambda m: (m, 0)),
            pl.BlockSpec((mt, 1), lambda m: (m, 0)),
        ],
        out_specs=pl.BlockSpec((2, mt, HF), lambda m: (0, m, 0)),
        out_shape=jax.ShapeDtypeStruct((2, N_NODES, HF), _f32),
    )(x, norm_out)


def _edge_agg_call(srcf, dst, h0v):
    mesh = plsc.VectorSubcoreMesh(core_axis_name="c", subcore_axis_name="s")

    @functools.partial(
        pl.kernel,
        mesh=mesh,
        out_type=jax.ShapeDtypeStruct((2, NPAD, HF), _f32),
        compiler_params=pltpu.CompilerParams(needs_layout_passes=False),
        scratch_types=[
            pltpu.VMEM((EPP,), _i32),
            pltpu.VMEM((EPP,), _i32),
            pltpu.VMEM((CH,), _i32),
            pltpu.VMEM((CH,), _i32),
            pltpu.VMEM((CH,), _i32),
            pltpu.VMEM((CH, HF), _f32),
            pltpu.VMEM((CH, HF), _f32),
            pltpu.VMEM_SHARED((NPAD, HF), _f32),
            pltpu.SemaphoreType.DMA,
            pltpu.SemaphoreType.DMA,
        ],
    )
    def agg_kernel(srcf_h, dst_h, h0v_h, agg_h,
                   g_v, dst_v, idx_a, idx_b, dsts_v, rows_a, rows_b,
                   acc_s, sem_a, sem_b):
        cid = lax.axis_index("c")
        sid = lax.axis_index("s")
        base = sid * EPP
        nb = sid * NTILE
        coff = cid * N_NODES

        pltpu.sync_copy(srcf_h.at[pl.ds(base, EPP)], g_v)
        pltpu.sync_copy(dst_h.at[pl.ds(base, EPP)], dst_v)

        z16 = jnp.zeros((16,), _f32)

        def zrow(i, c):
            for t in range(HF // 16):
                rows_a[i, pl.ds(t * 16, 16)] = z16
            return c

        lax.fori_loop(0, CH, zrow, 0)
        for q in range(NTILE // CH):
            pltpu.sync_copy(rows_a, acc_s.at[pl.ds(nb + q * CH, CH)])
        rem = NTILE - (NTILE // CH) * CH
        if rem:
            pltpu.sync_copy(rows_a.at[pl.ds(0, rem)],
                            acc_s.at[pl.ds(nb + (NTILE // CH) * CH, rem)])

        plsc.subcore_barrier()

        def build(flat, j, dref, off):
            for t in range(CH // 16):
                sl = pl.ds(t * 16, 16)
                dref[sl] = flat[pl.ds(j * CH + t * 16, 16)] + off

        def gather(j, idx, rows, sem):
            build(g_v, j, idx, coff)
            pltpu.async_copy(h0v_h.at[idx], rows, sem)

        def wait(rows, sem):
            pltpu.make_async_copy(h0v_h.at[idx_a], rows, sem).wait()

        def scatter(j, rows):
            build(dst_v, j, dsts_v, 0)
            pltpu.sync_copy(rows, acc_s.at[dsts_v], add=True)

        gather(0, idx_a, rows_a, sem_a)
        gather(1, idx_b, rows_b, sem_b)

        def pair_body(j2, c):
            ja = 2 * j2
            jb = ja + 1
            wait(rows_a, sem_a)
            scatter(ja, rows_a)
            gather(ja + 2, idx_a, rows_a, sem_a)
            wait(rows_b, sem_b)
            scatter(jb, rows_b)

            @pl.when(j2 < (NCH - 3) // 2)
            def _():
                gather(jb + 2, idx_b, rows_b, sem_b)

            return c

        lax.fori_loop(0, (NCH - 1) // 2, pair_body, 0)
        wait(rows_a, sem_a)
        scatter(NCH - 1, rows_a)

        plsc.subcore_barrier()

        @pl.when(cid == 0)
        def _():
            pltpu.sync_copy(acc_s.at[pl.ds(nb, NTILE)],
                            agg_h.at[0, pl.ds(nb, NTILE)])

        @pl.when(cid == 1)
        def _():
            pltpu.sync_copy(acc_s.at[pl.ds(nb, NTILE)],
                            agg_h.at[1, pl.ds(nb, NTILE)])

    return agg_kernel(srcf, dst, h0v)


def _dense_call(agg, w1r, b1, norm_in, u, norm_out, W2, b2):
    mt = 1024
    grid = NPAD // mt
    dn = (((1,), (0,)), ((), ()))
    dn_pool = (((0,), (0,)), ((), ()))
    prec = jax.lax.Precision.DEFAULT

    def body(agg_ref, w1_ref, b1_ref, nin_ref, u_ref, nout_ref, w2_ref,
             b2_ref, out_ref, pooled_ref):
        m = pl.program_id(0)

        @pl.when(m == 0)
        def _():
            pooled_ref[...] = jnp.zeros_like(pooled_ref)

        bf16 = jnp.bfloat16
        hid = lax.dot_general(agg_ref[0][...].astype(bf16),
                              w1_ref[0][...].astype(bf16), dn, precision=prec,
                              preferred_element_type=_f32)
        hid = hid + lax.dot_general(agg_ref[1][...].astype(bf16),
                                    w1_ref[1][...].astype(bf16), dn,
                                    precision=prec,
                                    preferred_element_type=_f32)
        hid = hid * nin_ref[...] + b1_ref[...]
        h1 = jnp.maximum(hid, 0.0)
        w = (u_ref[0] + u_ref[1]) * nout_ref[...]
        pooled_ref[...] += lax.dot_general(w, h1, dn_pool, precision=prec,
                                           preferred_element_type=_f32)

        @pl.when(m == grid - 1)
        def _():
            out_ref[...] = lax.dot_general(
                pooled_ref[...] * (1.0 / N_NODES), w2_ref[...], dn,
                precision=prec, preferred_element_type=_f32) + b2_ref[...]

    return pl.pallas_call(
        body,
        grid=(grid,),
        in_specs=[
            pl.BlockSpec((2, mt, HF), lambda m: (0, m, 0)),
            pl.BlockSpec((2, HF, H_FEATS), lambda m: (0, 0, 0)),
            pl.BlockSpec((1, H_FEATS), lambda m: (0, 0)),
            pl.BlockSpec((mt, 1), lambda m: (m, 0)),
            pl.BlockSpec((2, mt, 1), lambda m: (0, m, 0)),
            pl.BlockSpec((mt, 1), lambda m: (m, 0)),
            pl.BlockSpec((H_FEATS, NUM_CLASSES), lambda m: (0, 0)),
            pl.BlockSpec((1, NUM_CLASSES), lambda m: (0, 0)),
        ],
        out_specs=pl.BlockSpec((1, NUM_CLASSES), lambda m: (0, 0)),
        out_shape=jax.ShapeDtypeStruct((1, NUM_CLASSES), _f32),
        scratch_shapes=[pltpu.VMEM((1, H_FEATS), _f32)],
    )(agg, w1r, b1, norm_in, u, norm_out, W2, b2)


def kernel(x, edge_index, W1, b1, W2, b2):
    src = edge_index[0]
    dst = edge_index[1]

    norm_out_p, norm_in_p = _deg_norm_call(src, dst)
    u2 = _u_call(src, dst, norm_in_p)

    h0 = _scale_call(x, norm_out_p[:N_NODES, None])
    h0v = h0.reshape(2 * N_NODES, HF)

    pad = EPP - EPT
    spad = (jnp.arange(pad, dtype=_i32) * 125) % N_NODES
    dpad = N_NODES + (jnp.arange(pad, dtype=_i32) % (NPAD - N_NODES))
    srcp = jnp.concatenate(
        [src.reshape(16, EPT), jnp.broadcast_to(spad, (16, pad))],
        axis=1).reshape(-1)
    dstp = jnp.concatenate(
        [dst.reshape(16, EPT), jnp.broadcast_to(dpad, (16, pad))],
        axis=1).reshape(-1)
    agg = _edge_agg_call(srcp, dstp, h0v)

    w1r = W1.reshape(2, HF, H_FEATS)
    out = _dense_call(agg, w1r, b1.reshape(1, H_FEATS),
                      norm_in_p[:, None], u2[:, :, None], norm_out_p[:, None],
                      W2, b2.reshape(1, NUM_CLASSES))
    return out.reshape(NUM_CLASSES)


# speedup vs baseline: 1.0162x; 1.0103x over previous
import functools

import jax
import jax.numpy as jnp
from jax import lax
from jax.experimental import pallas as pl
from jax.experimental.pallas import tpu as pltpu
from jax.experimental.pallas import tpu_sc as plsc

N_NODES = 10000
N_EDGES = 160000
IN_FEATS = 256
H_FEATS = 1024
NUM_CLASSES = 128

NPAD = 10240
NTILE = NPAD // 16
HF = IN_FEATS // 2
EPT = N_EDGES // 16
CH = 96
NCH = 105
EPP = NCH * CH

_f32 = jnp.float32
_i32 = jnp.int32


def _rsqrt16(d):
    i = plsc.bitcast(d, _i32)
    i = 0x5F3759DF - lax.shift_right_logical(i, 1)
    y = plsc.bitcast(i, _f32)
    for _ in range(3):
        y = y * (1.5 - 0.5 * d * y * y)
    return jnp.where(d > 0.0, y, 0.0)


def _deg_norm_call(src, dst):
    mesh = plsc.VectorSubcoreMesh(core_axis_name="c", subcore_axis_name="s")

    @functools.partial(
        pl.kernel,
        mesh=mesh,
        out_type=[
            jax.ShapeDtypeStruct((NPAD,), _f32),
            jax.ShapeDtypeStruct((NPAD,), _f32),
        ],
        compiler_params=pltpu.CompilerParams(needs_layout_passes=False),
        scratch_types=[
            pltpu.VMEM((EPT,), _i32),
            pltpu.VMEM((NPAD,), _f32),
            pltpu.VMEM((NTILE,), _f32),
            pltpu.VMEM((NTILE,), _f32),
            pltpu.VMEM((NTILE,), _f32),
            pltpu.VMEM_SHARED((16, NPAD), _f32),
            pltpu.SemaphoreType.DMA,
            pltpu.SemaphoreType.DMA,
        ],
    )
    def deg_kernel(src_h, dst_h, nout_h, nin_h, idx_v, hist_v, tmp_v, tmp2_v,
                   acc_v, stage, sem_a, sem_b):
        cid = lax.axis_index("c")
        sid = lax.axis_index("s")
        base = sid * EPT
        nb = sid * NTILE

        @pl.when(cid == 0)
        def _():
            pltpu.sync_copy(src_h.at[pl.ds(base, EPT)], idx_v)

        @pl.when(cid == 1)
        def _():
            pltpu.sync_copy(dst_h.at[pl.ds(base, EPT)], idx_v)

        z16 = jnp.zeros((16,), _f32)
        ones16 = jnp.ones((16,), _f32)

        def zbody(i, c):
            hist_v[pl.ds(i * 16, 16)] = z16
            return c

        def ebody(i, c):
            idx = idx_v[pl.ds(i * 16, 16)]
            plsc.addupdate_scatter(hist_v, [idx], ones16)
            return c

        def abody_t(t, c):
            sl = pl.ds(t * 16, 16)
            acc_v[sl] = acc_v[sl] + tmp_v[sl]
            return c

        def abody_t2(t, c):
            sl = pl.ds(t * 16, 16)
            acc_v[sl] = acc_v[sl] + tmp2_v[sl]
            return c

        lax.fori_loop(0, NPAD // 16, zbody, 0)
        lax.fori_loop(0, EPT // 16, ebody, 0)

        pltpu.sync_copy(hist_v, stage.at[sid])
        plsc.subcore_barrier()

        bufs = [(tmp_v, sem_a, abody_t), (tmp2_v, sem_b, abody_t2)]
        pltpu.sync_copy(stage.at[0, pl.ds(nb, NTILE)], acc_v)
        pltpu.async_copy(stage.at[1, pl.ds(nb, NTILE)], tmp_v, sem_a)
        for k in range(1, 16):
            buf, sem, ab = bufs[(k - 1) % 2]
            nbuf, nsem, _ = bufs[k % 2]
            pltpu.make_async_copy(stage.at[k, pl.ds(nb, NTILE)], buf,
                                  sem).wait()
            if k < 15:
                pltpu.async_copy(stage.at[k + 1, pl.ds(nb, NTILE)], nbuf,
                                 nsem)
            lax.fori_loop(0, NTILE // 16, ab, 0)

        def rbody(t, c):
            sl = pl.ds(t * 16, 16)
            acc_v[sl] = _rsqrt16(acc_v[sl])
            return c

        lax.fori_loop(0, NTILE // 16, rbody, 0)

        @pl.when(cid == 0)
        def _():
            pltpu.sync_copy(acc_v, nout_h.at[pl.ds(nb, NTILE)])

        @pl.when(cid == 1)
        def _():
            pltpu.sync_copy(acc_v, nin_h.at[pl.ds(nb, NTILE)])

    return deg_kernel(src, dst)


def _u_call(src, dst, norm_in):
    EPT2 = N_EDGES // 32
    mesh = plsc.VectorSubcoreMesh(core_axis_name="c", subcore_axis_name="s")

    @functools.partial(
        pl.kernel,
        mesh=mesh,
        out_type=jax.ShapeDtypeStruct((2, NPAD), _f32),
        compiler_params=pltpu.CompilerParams(needs_layout_passes=False),
        scratch_types=[
            pltpu.VMEM((EPT2,), _i32),
            pltpu.VMEM((EPT2,), _i32),
            pltpu.VMEM((NPAD,), _f32),
            pltpu.VMEM((NPAD,), _f32),
            pltpu.VMEM((NTILE,), _f32),
            pltpu.VMEM((NTILE,), _f32),
            pltpu.VMEM((NTILE,), _f32),
            pltpu.VMEM_SHARED((16, NPAD), _f32),
            pltpu.SemaphoreType.DMA,
            pltpu.SemaphoreType.DMA,
        ],
    )
    def u_kernel(src_h, dst_h, nin_h, u_h, sv_v, dv_v, hist_v, normin_v,
                 tmp_v, tmp2_v, acc_v, stage, sem_a, sem_b):
        cid = lax.axis_index("c")
        sid = lax.axis_index("s")
        base = (cid * 16 + sid) * EPT2
        nb = sid * NTILE

        pltpu.sync_copy(src_h.at[pl.ds(base, EPT2)], sv_v)
        pltpu.sync_copy(dst_h.at[pl.ds(base, EPT2)], dv_v)
        pltpu.sync_copy(nin_h, normin_v)

        z16 = jnp.zeros((16,), _f32)

        def zbody(i, c):
            hist_v[pl.ds(i * 16, 16)] = z16
            return c

        lax.fori_loop(0, NPAD // 16, zbody, 0)

        def ubody(i, c):
            sl = pl.ds(i * 16, 16)
            vals = plsc.load_gather(normin_v, [dv_v[sl]])
            plsc.addupdate_scatter(hist_v, [sv_v[sl]], vals)
            return c

        lax.fori_loop(0, EPT2 // 16, ubody, 0)

        def abody_t(t, c):
            sl = pl.ds(t * 16, 16)
            acc_v[sl] = acc_v[sl] + tmp_v[sl]
            return c

        def abody_t2(t, c):
            sl = pl.ds(t * 16, 16)
            acc_v[sl] = acc_v[sl] + tmp2_v[sl]
            return c

        pltpu.sync_copy(hist_v, stage.at[sid])
        plsc.subcore_barrier()
        bufs = [(tmp_v, sem_a, abody_t), (tmp2_v, sem_b, abody_t2)]
        pltpu.sync_copy(stage.at[0, pl.ds(nb, NTILE)], acc_v)
        pltpu.async_copy(stage.at[1, pl.ds(nb, NTILE)], tmp_v, sem_a)
        for k in range(1, 16):
            buf, sem, ab = bufs[(k - 1) % 2]
            nbuf, nsem, _ = bufs[k % 2]
            pltpu.make_async_copy(stage.at[k, pl.ds(nb, NTILE)], buf,
                                  sem).wait()
            if k < 15:
                pltpu.async_copy(stage.at[k + 1, pl.ds(nb, NTILE)], nbuf,
                                 nsem)
            lax.fori_loop(0, NTILE // 16, ab, 0)

        @pl.when(cid == 0)
        def _():
            pltpu.sync_copy(acc_v, u_h.at[0, pl.ds(nb, NTILE)])

        @pl.when(cid == 1)
        def _():
            pltpu.sync_copy(acc_v, u_h.at[1, pl.ds(nb, NTILE)])

    return u_kernel(src, dst, norm_in)


def _scale_call(x, norm_out):
    mt = 1000

    def body(x_ref, n_ref, o_ref):
        sc = x_ref[...] * n_ref[...]
        o_ref[0] = sc[:, :HF]
        o_ref[1] = sc[:, HF:]

    return pl.pallas_call(
        body,
        grid=(N_NODES // mt,),
        in_specs=[
            pl.BlockSpec((mt, IN_FEATS), lambda m: (m, 0)),
            pl.BlockSpec((mt, 1), lambda m: (m, 0)),
        ],
        out_specs=pl.BlockSpec((2, mt, HF), lambda m: (0, m, 0)),
        out_shape=jax.ShapeDtypeStruct((2, N_NODES, HF), _f32),
    )(x, norm_out)


def _edge_agg_call(srcf, dst, h0v):
    mesh = plsc.VectorSubcoreMesh(core_axis_name="c", subcore_axis_name="s")

    @functools.partial(
        pl.kernel,
        mesh=mesh,
        out_type=jax.ShapeDtypeStruct((2, NPAD, HF), _f32),
        compiler_params=pltpu.CompilerParams(needs_layout_passes=False),
        scratch_types=[
            pltpu.VMEM((EPP,), _i32),
            pltpu.VMEM((EPP,), _i32),
            pltpu.VMEM((CH,), _i32),
            pltpu.VMEM((CH,), _i32),
            pltpu.VMEM((CH,), _i32),
            pltpu.VMEM((CH, HF), _f32),
            pltpu.VMEM((CH, HF), _f32),
            pltpu.VMEM_SHARED((NPAD, HF), _f32),
            pltpu.SemaphoreType.DMA,
            pltpu.SemaphoreType.DMA,
        ],
    )
    def agg_kernel(srcf_h, dst_h, h0v_h, agg_h,
                   g_v, dst_v, idx_a, idx_b, dsts_v, rows_a, rows_b,
                   acc_s, sem_a, sem_b):
        cid = lax.axis_index("c")
        sid = lax.axis_index("s")
        base = sid * EPP
        nb = sid * NTILE
        coff = cid * N_NODES

        pltpu.sync_copy(srcf_h.at[pl.ds(base, EPP)], g_v)
        pltpu.sync_copy(dst_h.at[pl.ds(base, EPP)], dst_v)

        z16 = jnp.zeros((16,), _f32)

        def zrow(i, c):
            for t in range(HF // 16):
                rows_a[i, pl.ds(t * 16, 16)] = z16
            return c

        lax.fori_loop(0, CH, zrow, 0)
        for q in range(NTILE // CH):
            pltpu.sync_copy(rows_a, acc_s.at[pl.ds(nb + q * CH, CH)])
        rem = NTILE - (NTILE // CH) * CH
        if rem:
            pltpu.sync_copy(rows_a.at[pl.ds(0, rem)],
                            acc_s.at[pl.ds(nb + (NTILE // CH) * CH, rem)])

        plsc.subcore_barrier()

        def build(flat, j, dref, off):
            for t in range(CH // 16):
                sl = pl.ds(t * 16, 16)
                dref[sl] = flat[pl.ds(j * CH + t * 16, 16)] + off

        def gather(j, idx, rows, sem):
            build(g_v, j, idx, coff)
            pltpu.async_copy(h0v_h.at[idx], rows, sem)

        def wait(rows, sem):
            pltpu.make_async_copy(h0v_h.at[idx_a], rows, sem).wait()

        def scatter(j, rows):
            build(dst_v, j, dsts_v, 0)
            pltpu.sync_copy(rows, acc_s.at[dsts_v], add=True)

        gather(0, idx_a, rows_a, sem_a)
        gather(1, idx_b, rows_b, sem_b)

        def pair_body(j2, c):
            ja = 2 * j2
            jb = ja + 1
            wait(rows_a, sem_a)
            scatter(ja, rows_a)
            gather(ja + 2, idx_a, rows_a, sem_a)
            wait(rows_b, sem_b)
            scatter(jb, rows_b)

            @pl.when(j2 < (NCH - 3) // 2)
            def _():
                gather(jb + 2, idx_b, rows_b, sem_b)

            return c

        lax.fori_loop(0, (NCH - 1) // 2, pair_body, 0)
        wait(rows_a, sem_a)
        scatter(NCH - 1, rows_a)

        plsc.subcore_barrier()

        @pl.when(cid == 0)
        def _():
            pltpu.sync_copy(acc_s.at[pl.ds(nb, NTILE)],
                            agg_h.at[0, pl.ds(nb, NTILE)])

        @pl.when(cid == 1)
        def _():
            pltpu.sync_copy(acc_s.at[pl.ds(nb, NTILE)],
                            agg_h.at[1, pl.ds(nb, NTILE)])

    return agg_kernel(srcf, dst, h0v)


def _dense_call(agg, w1r, b1, norm_in, u, norm_out, W2, b2):
    mt = 1024
    grid = NPAD // mt
    dn = (((1,), (0,)), ((), ()))
    dn_pool = (((0,), (0,)), ((), ()))
    prec = jax.lax.Precision.DEFAULT

    def body(agg_ref, w1_ref, b1_ref, nin_ref, u_ref, nout_ref, w2_ref,
             b2_ref, out_ref, pooled_ref):
        m = pl.program_id(0)

        @pl.when(m == 0)
        def _():
            pooled_ref[...] = jnp.zeros_like(pooled_ref)

        hid = lax.dot_general(agg_ref[0], w1_ref[0], dn, precision=prec,
                              preferred_element_type=_f32)
        hid = hid + lax.dot_general(agg_ref[1], w1_ref[1], dn, precision=prec,
                                    preferred_element_type=_f32)
        hid = hid * nin_ref[...] + b1_ref[...]
        h1 = jnp.maximum(hid, 0.0)
        w = (u_ref[0] + u_ref[1]) * nout_ref[...]
        pooled_ref[...] += lax.dot_general(w, h1, dn_pool, precision=prec,
                                           preferred_element_type=_f32)

        @pl.when(m == grid - 1)
        def _():
            out_ref[...] = lax.dot_general(
                pooled_ref[...] * (1.0 / N_NODES), w2_ref[...], dn,
                precision=prec, preferred_element_type=_f32) + b2_ref[...]

    return pl.pallas_call(
        body,
        grid=(grid,),
        in_specs=[
            pl.BlockSpec((2, mt, HF), lambda m: (0, m, 0)),
            pl.BlockSpec((2, HF, H_FEATS), lambda m: (0, 0, 0)),
            pl.BlockSpec((1, H_FEATS), lambda m: (0, 0)),
            pl.BlockSpec((mt, 1), lambda m: (m, 0)),
            pl.BlockSpec((2, mt, 1), lambda m: (0, m, 0)),
            pl.BlockSpec((mt, 1), lambda m: (m, 0)),
            pl.BlockSpec((H_FEATS, NUM_CLASSES), lambda m: (0, 0)),
            pl.BlockSpec((1, NUM_CLASSES), lambda m: (0, 0)),
        ],
        out_specs=pl.BlockSpec((1, NUM_CLASSES), lambda m: (0, 0)),
        out_shape=jax.ShapeDtypeStruct((1, NUM_CLASSES), _f32),
        scratch_shapes=[pltpu.VMEM((1, H_FEATS), _f32)],
    )(agg, w1r, b1, norm_in, u, norm_out, W2, b2)


def kernel(x, edge_index, W1, b1, W2, b2):
    src = edge_index[0]
    dst = edge_index[1]

    norm_out_p, norm_in_p = _deg_norm_call(src, dst)
    u2 = _u_call(src, dst, norm_in_p)

    h0 = _scale_call(x, norm_out_p[:N_NODES, None])
    h0v = h0.reshape(2 * N_NODES, HF)

    pad = EPP - EPT
    spad = (jnp.arange(pad, dtype=_i32) * 125) % N_NODES
    dpad = N_NODES + (jnp.arange(pad, dtype=_i32) % (NPAD - N_NODES))
    srcp = jnp.concatenate(
        [src.reshape(16, EPT), jnp.broadcast_to(spad, (16, pad))],
        axis=1).reshape(-1)
    dstp = jnp.concatenate(
        [dst.reshape(16, EPT), jnp.broadcast_to(dpad, (16, pad))],
        axis=1).reshape(-1)
    agg = _edge_agg_call(srcp, dstp, h0v)

    w1r = W1.reshape(2, HF, H_FEATS)
    out = _dense_call(agg, w1r, b1.reshape(1, H_FEATS),
                      norm_in_p[:, None], u2[:, :, None], norm_out_p[:, None],
                      W2, b2.reshape(1, NUM_CLASSES))
    return out.reshape(NUM_CLASSES)
